# Initial kernel scaffold; baseline (speedup 1.0000x reference)
import jax
import jax.numpy as jnp
from jax.experimental import pallas as pl


def kernel(points, weight, bias):
    raise NotImplementedError("write your pallas kernel here")


# speedup vs baseline: 1.1434x; 1.1434x over previous
import functools

import jax
import jax.numpy as jnp
from jax.experimental import pallas as pl
from jax.experimental.pallas import tpu as pltpu

_LANE = 128
_NEG_INF = float("-inf")


def _ceil_to(x, m):
    return (x + m - 1) // m * m


def _pool_classify_kernel(x_ref, w_ref, b_ref, o_ref, acc_ref, *, n_chunks,
                          d_valid):
    x = x_ref[...]
    live = jnp.any(x != 0.0, axis=-1, keepdims=True)
    chunk_max = jnp.max(jnp.where(live, x, _NEG_INF), axis=1)

    step = pl.program_id(1)
    @pl.when(step == 0)
    def _seed():
        acc_ref[...] = chunk_max

    @pl.when(step > 0)
    def _accumulate():
        acc_ref[...] = jnp.maximum(acc_ref[...], chunk_max)

    @pl.when(step == n_chunks - 1)
    def _classify():
        pooled = acc_ref[...]
        if d_valid is not None:
            lane = jax.lax.broadcasted_iota(jnp.int32, pooled.shape, 1)
            pooled = jnp.where(lane < d_valid, pooled, 0.0)
        o_ref[...] = (
            jnp.dot(pooled, w_ref[...], preferred_element_type=jnp.float32)
            + b_ref[...]
        )


def _dense_kernel(x_ref, w_ref, b_ref, o_ref):
    o_ref[...] = (
        jnp.dot(x_ref[...], w_ref[...], preferred_element_type=jnp.float32)
        + b_ref[...]
    )


def _prep_params(weight, bias, dp, cp):
    n_classes, d = weight.shape
    wt = jnp.zeros((dp, cp), jnp.float32).at[:d, :n_classes].set(
        weight.T.astype(jnp.float32))
    bb = jnp.zeros((1, cp), jnp.float32).at[:, :n_classes].set(
        bias.astype(jnp.float32))
    return wt, bb


def kernel(points, weight, bias):
    n_classes, d = weight.shape
    dp = _ceil_to(d, _LANE)
    cp = _ceil_to(n_classes, _LANE)
    wt, bb = _prep_params(weight, bias, dp, cp)
    cparams = pltpu.CompilerParams(
        dimension_semantics=("parallel", "arbitrary"),
        vmem_limit_bytes=32 * 1024 * 1024,
    )

    if points.ndim == 2:
        B = points.shape[0]
        tb = min(256, _ceil_to(B, 8))
        bp = _ceil_to(B, tb)
        x = points.astype(jnp.float32)
        if (bp, dp) != x.shape:
            x = jnp.zeros((bp, dp), jnp.float32).at[:B, :d].set(x)
        out = pl.pallas_call(
            _dense_kernel,
            out_shape=jax.ShapeDtypeStruct((bp, cp), jnp.float32),
            grid=(bp // tb, 1),
            in_specs=[
                pl.BlockSpec((tb, dp), lambda i, j: (i, 0)),
                pl.BlockSpec((dp, cp), lambda i, j: (0, 0)),
                pl.BlockSpec((1, cp), lambda i, j: (0, 0)),
            ],
            out_specs=pl.BlockSpec((tb, cp), lambda i, j: (i, 0)),
            compiler_params=cparams,
        )(x, wt, bb)
        return out[:B, :n_classes]

    B, N, _ = points.shape
    tb = 8 if B % 8 == 0 else min(8, _ceil_to(B, 8))
    bp = _ceil_to(B, tb)
    max_chunk_elems = (8 * 1024 * 1024) // 4
    tn = max(8, min(_ceil_to(N, 8), (max_chunk_elems // (tb * dp)) // 8 * 8))
    np_ = _ceil_to(N, tn)

    x = points.astype(jnp.float32)
    if (bp, np_, dp) != x.shape:
        x = jnp.zeros((bp, np_, dp), jnp.float32).at[:B, :N, :d].set(x)

    n_chunks = np_ // tn
    body = functools.partial(
        _pool_classify_kernel,
        n_chunks=n_chunks,
        d_valid=d if dp != d else None,
    )
    out = pl.pallas_call(
        body,
        out_shape=jax.ShapeDtypeStruct((bp, cp), jnp.float32),
        grid=(bp // tb, n_chunks),
        in_specs=[
            pl.BlockSpec((tb, tn, dp), lambda i, k: (i, k, 0)),
            pl.BlockSpec((dp, cp), lambda i, k: (0, 0)),
            pl.BlockSpec((1, cp), lambda i, k: (0, 0)),
        ],
        out_specs=pl.BlockSpec((tb, cp), lambda i, k: (i, 0)),
        scratch_shapes=[pltpu.VMEM((tb, dp), jnp.float32)],
        compiler_params=cparams,
    )(x, wt, bb)
    return out[:B, :n_classes]


# speedup vs baseline: 1.1600x; 1.0145x over previous
import functools

import jax
import jax.numpy as jnp
from jax.experimental import pallas as pl
from jax.experimental.pallas import tpu as pltpu

_LANE = 128
_NEG_INF = float("-inf")


def _ceil_to(x, m):
    return (x + m - 1) // m * m


def _pool_classify_kernel(x_ref, w_ref, b_ref, o_ref, acc_ref, *, n_chunks,
                          d_valid):
    x = x_ref[...]
    live = jnp.any(x != 0.0, axis=-1, keepdims=True)
    chunk_max = jnp.max(jnp.where(live, x, _NEG_INF), axis=1)

    step = pl.program_id(1)
    @pl.when(step == 0)
    def _seed():
        acc_ref[...] = chunk_max

    @pl.when(step > 0)
    def _accumulate():
        acc_ref[...] = jnp.maximum(acc_ref[...], chunk_max)

    @pl.when(step == n_chunks - 1)
    def _classify():
        pooled = acc_ref[...]
        if d_valid is not None:
            lane = jax.lax.broadcasted_iota(jnp.int32, pooled.shape, 1)
            pooled = jnp.where(lane < d_valid, pooled, 0.0)
        y = jax.lax.dot_general(
            pooled, w_ref[...],
            dimension_numbers=(((1,), (1,)), ((), ())),
            preferred_element_type=jnp.float32,
        )
        o_ref[...] = y + b_ref[...]


def _dense_kernel(x_ref, w_ref, b_ref, o_ref):
    o_ref[...] = (
        jnp.dot(x_ref[...], w_ref[...], preferred_element_type=jnp.float32)
        + b_ref[...]
    )


def _prep_params(weight, bias, dp, cp):
    n_classes, d = weight.shape
    wt = jnp.zeros((dp, cp), jnp.float32).at[:d, :n_classes].set(
        weight.T.astype(jnp.float32))
    bb = jnp.zeros((1, cp), jnp.float32).at[:, :n_classes].set(
        bias.astype(jnp.float32))
    return wt, bb


def kernel(points, weight, bias):
    n_classes, d = weight.shape
    dp = _ceil_to(d, _LANE)
    cp = _ceil_to(n_classes, _LANE)
    wt, bb = _prep_params(weight, bias, dp, cp)
    cparams = pltpu.CompilerParams(
        dimension_semantics=("parallel", "arbitrary"),
        vmem_limit_bytes=32 * 1024 * 1024,
    )

    if points.ndim == 2:
        B = points.shape[0]
        tb = min(256, _ceil_to(B, 8))
        bp = _ceil_to(B, tb)
        x = points.astype(jnp.float32)
        if (bp, dp) != x.shape:
            x = jnp.zeros((bp, dp), jnp.float32).at[:B, :d].set(x)
        out = pl.pallas_call(
            _dense_kernel,
            out_shape=jax.ShapeDtypeStruct((bp, cp), jnp.float32),
            grid=(bp // tb, 1),
            in_specs=[
                pl.BlockSpec((tb, dp), lambda i, j: (i, 0)),
                pl.BlockSpec((dp, cp), lambda i, j: (0, 0)),
                pl.BlockSpec((1, cp), lambda i, j: (0, 0)),
            ],
            out_specs=pl.BlockSpec((tb, cp), lambda i, j: (i, 0)),
            compiler_params=cparams,
        )(x, wt, bb)
        return out[:B, :n_classes]

    B, N, _ = points.shape
    tb = 8 if B % 8 == 0 else min(8, _ceil_to(B, 8))
    bp = _ceil_to(B, tb)
    max_chunk_elems = (8 * 1024 * 1024) // 4
    tn = max(8, min(_ceil_to(N, 8), (max_chunk_elems // (tb * dp)) // 8 * 8))
    np_ = _ceil_to(N, tn)

    x = points.astype(jnp.float32)
    if (bp, np_, dp) != x.shape:
        x = jnp.zeros((bp, np_, dp), jnp.float32).at[:B, :N, :d].set(x)

    c8 = _ceil_to(n_classes, 8)
    w = weight.astype(jnp.float32)
    bvec = bias.astype(jnp.float32).reshape(1, n_classes)
    if (c8, dp) != w.shape:
        w = jnp.zeros((c8, dp), jnp.float32).at[:n_classes, :d].set(w)
        bvec = jnp.zeros((1, c8), jnp.float32).at[:, :n_classes].set(bvec)

    n_chunks = np_ // tn
    body = functools.partial(
        _pool_classify_kernel,
        n_chunks=n_chunks,
        d_valid=d if dp != d else None,
    )
    out = pl.pallas_call(
        body,
        out_shape=jax.ShapeDtypeStruct((bp, c8), jnp.float32),
        grid=(bp // tb, n_chunks),
        in_specs=[
            pl.BlockSpec((tb, tn, dp), lambda i, k: (i, k, 0)),
            pl.BlockSpec((c8, dp), lambda i, k: (0, 0)),
            pl.BlockSpec((1, c8), lambda i, k: (0, 0)),
        ],
        out_specs=pl.BlockSpec((tb, c8), lambda i, k: (i, 0)),
        scratch_shapes=[pltpu.VMEM((tb, dp), jnp.float32)],
        compiler_params=cparams,
    )(x, w, bvec)
    return out[:B, :n_classes]
